# Initial kernel scaffold; baseline (speedup 1.0000x reference)
import functools

import jax
import jax.numpy as jnp
from jax import lax
from jax.experimental import pallas as pl
from jax.experimental.pallas import tpu as pltpu
from jax.experimental.pallas import tpu_sc as plsc


def kernel(sequence, pos_table):
    B, L = sequence.shape
    D = pos_table.shape[1]
    table = pos_table[:L]

    info = plsc.get_sparse_core_info()
    NW = info.num_cores * info.num_subcores
    rows_per_w = B // NW
    REP = 8
    n_chunks = rows_per_w // REP

    mesh = plsc.VectorSubcoreMesh(core_axis_name="c", subcore_axis_name="s")

    @functools.partial(
        pl.kernel,
        mesh=mesh,
        out_type=jax.ShapeDtypeStruct((B, L, D), jnp.float32),
        scratch_types=[
            pltpu.VMEM((REP, L, D), jnp.float32),
            pltpu.SemaphoreType.DMA,
        ],
    )
    def pe_kernel(table_hbm, out_hbm, rep_v, sem):
        wid = lax.axis_index("s") * info.num_cores + lax.axis_index("c")
        base = wid * rows_per_w
        for j in range(REP):
            pltpu.sync_copy(table_hbm, rep_v.at[j])
        copies = [
            pltpu.async_copy(rep_v, out_hbm.at[pl.ds(base + i * REP, REP)], sem)
            for i in range(n_chunks)
        ]
        for c in copies:
            c.wait()

    return pe_kernel(table)


# speedup vs baseline: 5.6378x; 5.6378x over previous
import functools

import jax
import jax.numpy as jnp
from jax import lax
from jax.experimental import pallas as pl
from jax.experimental.pallas import tpu as pltpu
from jax.experimental.pallas import tpu_sc as plsc


def kernel(sequence, pos_table):
    B, L = sequence.shape
    D = pos_table.shape[1]
    row = L * D
    table = pos_table[:L].reshape(row)

    info = plsc.get_sparse_core_info()
    NW = info.num_cores * info.num_subcores
    rows_per_w = B // NW
    REP = 8
    n_chunks = rows_per_w // REP
    chunk = REP * row

    mesh = plsc.VectorSubcoreMesh(core_axis_name="c", subcore_axis_name="s")

    @functools.partial(
        pl.kernel,
        mesh=mesh,
        out_type=jax.ShapeDtypeStruct((B * row,), jnp.float32),
        scratch_types=[
            pltpu.VMEM((chunk,), jnp.float32),
            pltpu.SemaphoreType.DMA,
        ],
    )
    def pe_kernel(table_hbm, out_hbm, rep_v, sem):
        wid = lax.axis_index("s") * info.num_cores + lax.axis_index("c")
        base = wid * rows_per_w * row
        for j in range(REP):
            pltpu.sync_copy(table_hbm, rep_v.at[pl.ds(j * row, row)])
        copies = [
            pltpu.async_copy(rep_v, out_hbm.at[pl.ds(base + i * chunk, chunk)], sem)
            for i in range(n_chunks)
        ]
        for c in copies:
            c.wait()

    return pe_kernel(table).reshape(B, L, D)
